# baseline (speedup 1.0000x reference)
import functools

import jax
import jax.numpy as jnp
from jax import lax
from jax.experimental import pallas as pl
from jax.experimental.pallas import tpu as pltpu
from jax.experimental.pallas import tpu_sc as plsc

N_NODES = 10000
N_FEAT = 128
N_EDGES = 320000
N_CLASS = 40

NC = 2
NS = 16
EDGES_PER_TILE = N_EDGES // (NC * NS)
CHUNK = 80
NCHUNK = EDGES_PER_TILE // CHUNK
NSEG = 5
SEGCHUNK = NCHUNK // NSEG
ROWS_PER_TILE = 624
TAIL_ROWS = N_NODES - NS * ROWS_PER_TILE
ZROWS = 24


def _sc_segment_sum(x, idx6):
    mesh = plsc.VectorSubcoreMesh(core_axis_name="c", subcore_axis_name="s")

    @functools.partial(
        pl.kernel,
        mesh=mesh,
        out_type=jax.ShapeDtypeStruct((NC, N_NODES, N_FEAT), jnp.float32),
        scratch_types=[
            pltpu.VMEM((2, SEGCHUNK, CHUNK), jnp.int32),
            pltpu.VMEM((2, SEGCHUNK, CHUNK), jnp.int32),
            pltpu.VMEM((CHUNK, N_FEAT), jnp.float32),
            pltpu.VMEM((CHUNK, N_FEAT), jnp.float32),
            pltpu.VMEM((CHUNK, N_FEAT), jnp.float32),
            pltpu.VMEM_SHARED((N_NODES, N_FEAT), jnp.float32),
            pltpu.SemaphoreType.DMA,
            pltpu.SemaphoreType.DMA,
            pltpu.SemaphoreType.DMA,
            pltpu.SemaphoreType.DMA,
            pltpu.SemaphoreType.DMA,
        ],
    )
    def seg_kernel(x_hbm, idx_hbm, out_hbm,
                   srcv, dstv, bufa, bufb, bufc, aggs,
                   sema, semb, semc, semi0, semi1):
        c = lax.axis_index("c")
        s = lax.axis_index("s")
        row0 = s * ROWS_PER_TILE

        @pl.when(c == 0)
        def _():
            pltpu.sync_copy(x_hbm.at[pl.ds(row0, ROWS_PER_TILE)],
                            aggs.at[pl.ds(row0, ROWS_PER_TILE)])

            @pl.when(s == 0)
            def _():
                pltpu.sync_copy(x_hbm.at[pl.ds(NS * ROWS_PER_TILE, TAIL_ROWS)],
                                aggs.at[pl.ds(NS * ROWS_PER_TILE, TAIL_ROWS)])

        @pl.when(c == 1)
        def _():
            def zstore(k, carry):
                r = k // (N_FEAT // 16)
                col = (k % (N_FEAT // 16)) * 16
                bufa[r, pl.ds(col, 16)] = jnp.zeros((16,), jnp.float32)
                return carry
            lax.fori_loop(0, ZROWS * (N_FEAT // 16), zstore, 0)

            def zcopy(i, carry):
                pltpu.sync_copy(bufa.at[pl.ds(0, ZROWS)],
                                aggs.at[pl.ds(row0 + i * ZROWS, ZROWS)])
                return carry
            lax.fori_loop(0, ROWS_PER_TILE // ZROWS, zcopy, 0)

            @pl.when(s == 0)
            def _():
                pltpu.sync_copy(bufa.at[pl.ds(0, TAIL_ROWS)],
                                aggs.at[pl.ds(NS * ROWS_PER_TILE, TAIL_ROWS)])

        pltpu.sync_copy(idx_hbm.at[0, c, s, 0], srcv.at[0])
        pltpu.sync_copy(idx_hbm.at[1, c, s, 0], dstv.at[0])
        pltpu.make_async_copy(idx_hbm.at[0, c, s, 1], srcv.at[1], semi1).start()
        pltpu.make_async_copy(idx_hbm.at[1, c, s, 1], dstv.at[1], semi1).start()
        plsc.subcore_barrier()

        bufs = (bufa, bufb, bufc)
        sems = (sema, semb, semc)

        def seg(g, carry):
            p = g % 2
            sv = srcv.at[p]
            dv = dstv.at[p]

            @pl.when((g > 0) & (p == 0))
            def _():
                pltpu.make_async_copy(idx_hbm.at[0, c, s, g], sv, semi0).wait()
                pltpu.make_async_copy(idx_hbm.at[1, c, s, g], dv, semi0).wait()

            @pl.when((g > 0) & (p == 1))
            def _():
                pltpu.make_async_copy(idx_hbm.at[0, c, s, g], sv, semi1).wait()
                pltpu.make_async_copy(idx_hbm.at[1, c, s, g], dv, semi1).wait()

            pltpu.make_async_copy(x_hbm.at[sv.at[0]], bufs[0], sems[0]).start()
            pltpu.make_async_copy(x_hbm.at[sv.at[1]], bufs[1], sems[1]).start()
            for j in range(SEGCHUNK):
                b = j % 3
                pltpu.make_async_copy(x_hbm.at[sv.at[j]], bufs[b], sems[b]).wait()
                pltpu.make_async_copy(
                    bufs[b], aggs.at[dv.at[j]], sems[b]).start(add=True)
                if j + 2 < SEGCHUNK:
                    b2 = (j + 2) % 3
                    if j >= 1:
                        pltpu.make_async_copy(
                            bufs[b2], aggs.at[dv.at[j - 1]], sems[b2]).wait()
                    pltpu.make_async_copy(
                        x_hbm.at[sv.at[j + 2]], bufs[b2], sems[b2]).start()
            for j in range(SEGCHUNK - 3, SEGCHUNK):
                b = j % 3
                pltpu.make_async_copy(
                    bufs[b], aggs.at[dv.at[j]], sems[b]).wait()

            @pl.when((g + 2 < NSEG) & (p == 0))
            def _():
                pltpu.make_async_copy(idx_hbm.at[0, c, s, g + 2], srcv.at[p], semi0).start()
                pltpu.make_async_copy(idx_hbm.at[1, c, s, g + 2], dstv.at[p], semi0).start()

            @pl.when((g + 2 < NSEG) & (p == 1))
            def _():
                pltpu.make_async_copy(idx_hbm.at[0, c, s, g + 2], srcv.at[p], semi1).start()
                pltpu.make_async_copy(idx_hbm.at[1, c, s, g + 2], dstv.at[p], semi1).start()
            return carry
        lax.fori_loop(0, NSEG, seg, 0)

        plsc.subcore_barrier()
        pltpu.sync_copy(aggs.at[pl.ds(row0, ROWS_PER_TILE)],
                        out_hbm.at[c, pl.ds(row0, ROWS_PER_TILE)])

        @pl.when(s == 0)
        def _():
            pltpu.sync_copy(aggs.at[pl.ds(NS * ROWS_PER_TILE, TAIL_ROWS)],
                            out_hbm.at[c, pl.ds(NS * ROWS_PER_TILE, TAIL_ROWS)])

    return seg_kernel(x, idx6)


def _tc_mlp(parts, w1t, b1, gamma, beta, w2t, b2, wfct, bfc):
    BLK = 2000
    G = N_NODES // BLK
    inv_n = 1.0 / N_NODES

    def k(p_r, w1_r, b1_r, g_r, be_r, w2_r, b2_r, wf_r, bf_r,
          out_r, h1_s, sums_s):
        t = pl.program_id(0)
        i = pl.program_id(1)

        @pl.when(t == 0)
        def _():
            h = p_r[0] + p_r[1]
            h1 = lax.dot_general(h, w1_r[...], (((1,), (1,)), ((), ())),
                                 preferred_element_type=jnp.float32) + b1_r[...]
            h1 = jnp.maximum(h1, 0.0)
            h1_s[pl.ds(i * BLK, BLK), :] = h1

            @pl.when(i == 0)
            def _():
                sums_s[...] = jnp.zeros_like(sums_s)
            sums_s[0:1, :] += jnp.sum(h1, axis=0, keepdims=True)
            sums_s[1:2, :] += jnp.sum(h1 * h1, axis=0, keepdims=True)

        @pl.when(t == 1)
        def _():
            mean = sums_s[0:1, :] * inv_n
            var = sums_s[1:2, :] * inv_n - mean * mean
            sc = g_r[...] * lax.rsqrt(var + 1e-5)
            sh = be_r[...] - mean * sc
            ws = lax.dot_general(w2_r[...], wf_r[...], (((0,), (1,)), ((), ())),
                                 preferred_element_type=jnp.float32)
            bs = lax.dot_general(b2_r[...], wf_r[...], (((1,), (1,)), ((), ())),
                                 preferred_element_type=jnp.float32) + bf_r[...]
            hn = h1_s[pl.ds(i * BLK, BLK), :] * sc + sh
            out_r[...] = jnp.dot(hn, ws, preferred_element_type=jnp.float32) + bs

    def part_map(t, i):
        return (0, jnp.where(t == 0, i, 0), 0)

    return pl.pallas_call(
        k,
        grid=(2, G),
        in_specs=[
            pl.BlockSpec((NC, BLK, N_FEAT), part_map),
            pl.BlockSpec((N_FEAT, N_FEAT), lambda t, i: (0, 0)),
            pl.BlockSpec((1, N_FEAT), lambda t, i: (0, 0)),
            pl.BlockSpec((1, N_FEAT), lambda t, i: (0, 0)),
            pl.BlockSpec((1, N_FEAT), lambda t, i: (0, 0)),
            pl.BlockSpec((N_FEAT, N_FEAT), lambda t, i: (0, 0)),
            pl.BlockSpec((1, N_FEAT), lambda t, i: (0, 0)),
            pl.BlockSpec((N_CLASS, N_FEAT), lambda t, i: (0, 0)),
            pl.BlockSpec((1, N_CLASS), lambda t, i: (0, 0)),
        ],
        out_specs=pl.BlockSpec((BLK, N_CLASS), lambda t, i: (i, 0)),
        out_shape=jax.ShapeDtypeStruct((N_NODES, N_CLASS), jnp.float32),
        scratch_shapes=[
            pltpu.VMEM((N_NODES, N_FEAT), jnp.float32),
            pltpu.VMEM((2, N_FEAT), jnp.float32),
        ],
    )(parts, w1t, b1, gamma, beta, w2t, b2, wfct, bfc)


def kernel(x, edge_index, W1, b1, gamma, beta, W2, b2, Wfc, bfc):
    idx6 = edge_index.astype(jnp.int32).reshape(2, NC, NS, NSEG, SEGCHUNK, CHUNK)
    parts = _sc_segment_sum(x, idx6)
    out = _tc_mlp(parts, W1, b1.reshape(1, N_FEAT),
                  gamma.reshape(1, N_FEAT), beta.reshape(1, N_FEAT),
                  W2, b2.reshape(1, N_FEAT), Wfc, bfc.reshape(1, N_CLASS))
    return out


# speedup vs baseline: 1.0003x; 1.0003x over previous
import functools

import jax
import jax.numpy as jnp
from jax import lax
from jax.experimental import pallas as pl
from jax.experimental.pallas import tpu as pltpu
from jax.experimental.pallas import tpu_sc as plsc

N_NODES = 10000
N_FEAT = 128
N_EDGES = 320000
N_CLASS = 40

NC = 2
NS = 16
EDGES_PER_TILE = N_EDGES // (NC * NS)
CHUNK = 80
NCHUNK = EDGES_PER_TILE // CHUNK
NSEG = 5
SEGCHUNK = NCHUNK // NSEG
ROWS_PER_TILE = 624
TAIL_ROWS = N_NODES - NS * ROWS_PER_TILE
ZROWS = 24


def _sc_segment_sum(x, idx6):
    mesh = plsc.VectorSubcoreMesh(core_axis_name="c", subcore_axis_name="s")

    @functools.partial(
        pl.kernel,
        mesh=mesh,
        out_type=jax.ShapeDtypeStruct((NC, N_NODES, N_FEAT), jnp.float32),
        scratch_types=[
            pltpu.VMEM((2, SEGCHUNK, CHUNK), jnp.int32),
            pltpu.VMEM((2, SEGCHUNK, CHUNK), jnp.int32),
            pltpu.VMEM((CHUNK, N_FEAT), jnp.float32),
            pltpu.VMEM((CHUNK, N_FEAT), jnp.float32),
            pltpu.VMEM((CHUNK, N_FEAT), jnp.float32),
            pltpu.VMEM_SHARED((N_NODES, N_FEAT), jnp.float32),
            pltpu.SemaphoreType.DMA,
            pltpu.SemaphoreType.DMA,
            pltpu.SemaphoreType.DMA,
            pltpu.SemaphoreType.DMA,
            pltpu.SemaphoreType.DMA,
        ],
    )
    def seg_kernel(x_hbm, idx_hbm, out_hbm,
                   srcv, dstv, bufa, bufb, bufc, aggs,
                   sema, semb, semc, semi0, semi1):
        c = lax.axis_index("c")
        s = lax.axis_index("s")
        row0 = s * ROWS_PER_TILE

        @pl.when(c == 0)
        def _():
            pltpu.sync_copy(x_hbm.at[pl.ds(row0, ROWS_PER_TILE)],
                            aggs.at[pl.ds(row0, ROWS_PER_TILE)])

            @pl.when(s == 0)
            def _():
                pltpu.sync_copy(x_hbm.at[pl.ds(NS * ROWS_PER_TILE, TAIL_ROWS)],
                                aggs.at[pl.ds(NS * ROWS_PER_TILE, TAIL_ROWS)])

        @pl.when(c == 1)
        def _():
            def zstore(k, carry):
                r = k // (N_FEAT // 16)
                col = (k % (N_FEAT // 16)) * 16
                bufa[r, pl.ds(col, 16)] = jnp.zeros((16,), jnp.float32)
                return carry
            lax.fori_loop(0, ZROWS * (N_FEAT // 16), zstore, 0)

            def zcopy(i, carry):
                pltpu.sync_copy(bufa.at[pl.ds(0, ZROWS)],
                                aggs.at[pl.ds(row0 + i * ZROWS, ZROWS)])
                return carry
            lax.fori_loop(0, ROWS_PER_TILE // ZROWS, zcopy, 0)

            @pl.when(s == 0)
            def _():
                pltpu.sync_copy(bufa.at[pl.ds(0, TAIL_ROWS)],
                                aggs.at[pl.ds(NS * ROWS_PER_TILE, TAIL_ROWS)])

        pltpu.sync_copy(idx_hbm.at[0, c, s, 0], srcv.at[0])
        pltpu.sync_copy(idx_hbm.at[1, c, s, 0], dstv.at[0])
        pltpu.make_async_copy(idx_hbm.at[0, c, s, 1], srcv.at[1], semi1).start()
        pltpu.make_async_copy(idx_hbm.at[1, c, s, 1], dstv.at[1], semi1).start()
        plsc.subcore_barrier()

        bufs = (bufa, bufb, bufc)
        sems = (sema, semb, semc)

        def seg(g, carry):
            p = g % 2
            sv = srcv.at[p]
            dv = dstv.at[p]

            @pl.when((g > 0) & (p == 0))
            def _():
                pltpu.make_async_copy(idx_hbm.at[0, c, s, g], sv, semi0).wait()
                pltpu.make_async_copy(idx_hbm.at[1, c, s, g], dv, semi0).wait()

            @pl.when((g > 0) & (p == 1))
            def _():
                pltpu.make_async_copy(idx_hbm.at[0, c, s, g], sv, semi1).wait()
                pltpu.make_async_copy(idx_hbm.at[1, c, s, g], dv, semi1).wait()

            pltpu.make_async_copy(x_hbm.at[sv.at[0]], bufs[0], sems[0]).start()
            pltpu.make_async_copy(x_hbm.at[sv.at[1]], bufs[1], sems[1]).start()
            for j in range(SEGCHUNK):
                b = j % 3
                pltpu.make_async_copy(x_hbm.at[sv.at[j]], bufs[b], sems[b]).wait()
                pltpu.make_async_copy(
                    bufs[b], aggs.at[dv.at[j]], sems[b]).start(add=True)
                if j + 2 < SEGCHUNK:
                    b2 = (j + 2) % 3
                    if j >= 1:
                        pltpu.make_async_copy(
                            bufs[b2], aggs.at[dv.at[j - 1]], sems[b2]).wait()
                    pltpu.make_async_copy(
                        x_hbm.at[sv.at[j + 2]], bufs[b2], sems[b2]).start()
            for j in range(SEGCHUNK - 3, SEGCHUNK):
                b = j % 3
                pltpu.make_async_copy(
                    bufs[b], aggs.at[dv.at[j]], sems[b]).wait()

            @pl.when((g + 2 < NSEG) & (p == 0))
            def _():
                pltpu.make_async_copy(idx_hbm.at[0, c, s, g + 2], srcv.at[p], semi0).start()
                pltpu.make_async_copy(idx_hbm.at[1, c, s, g + 2], dstv.at[p], semi0).start()

            @pl.when((g + 2 < NSEG) & (p == 1))
            def _():
                pltpu.make_async_copy(idx_hbm.at[0, c, s, g + 2], srcv.at[p], semi1).start()
                pltpu.make_async_copy(idx_hbm.at[1, c, s, g + 2], dstv.at[p], semi1).start()
            return carry
        lax.fori_loop(0, NSEG, seg, 0)

        plsc.subcore_barrier()
        pltpu.sync_copy(aggs.at[pl.ds(row0, ROWS_PER_TILE)],
                        out_hbm.at[c, pl.ds(row0, ROWS_PER_TILE)])

        @pl.when(s == 0)
        def _():
            pltpu.sync_copy(aggs.at[pl.ds(NS * ROWS_PER_TILE, TAIL_ROWS)],
                            out_hbm.at[c, pl.ds(NS * ROWS_PER_TILE, TAIL_ROWS)])

    return seg_kernel(x, idx6)


def _tc_mlp(parts, w1t, b1, gamma, beta, w2t, b2, wfct, bfc):
    BLK = 5000
    G = N_NODES // BLK
    inv_n = 1.0 / N_NODES

    def k(p_r, w1_r, b1_r, g_r, be_r, w2_r, b2_r, wf_r, bf_r,
          out_r, h1_s, sums_s):
        t = pl.program_id(0)
        i = pl.program_id(1)

        @pl.when(t == 0)
        def _():
            h = p_r[0] + p_r[1]
            h1 = lax.dot_general(h, w1_r[...], (((1,), (1,)), ((), ())),
                                 preferred_element_type=jnp.float32) + b1_r[...]
            h1 = jnp.maximum(h1, 0.0)
            h1_s[pl.ds(i * BLK, BLK), :] = h1

            @pl.when(i == 0)
            def _():
                sums_s[...] = jnp.zeros_like(sums_s)
            sums_s[0:1, :] += jnp.sum(h1, axis=0, keepdims=True)
            sums_s[1:2, :] += jnp.sum(h1 * h1, axis=0, keepdims=True)

        @pl.when(t == 1)
        def _():
            mean = sums_s[0:1, :] * inv_n
            var = sums_s[1:2, :] * inv_n - mean * mean
            sc = g_r[...] * lax.rsqrt(var + 1e-5)
            sh = be_r[...] - mean * sc
            ws = lax.dot_general(w2_r[...], wf_r[...], (((0,), (1,)), ((), ())),
                                 preferred_element_type=jnp.float32)
            bs = lax.dot_general(b2_r[...], wf_r[...], (((1,), (1,)), ((), ())),
                                 preferred_element_type=jnp.float32) + bf_r[...]
            hn = h1_s[pl.ds(i * BLK, BLK), :] * sc + sh
            out_r[...] = jnp.dot(hn, ws, preferred_element_type=jnp.float32) + bs

    def part_map(t, i):
        return (0, jnp.where(t == 0, i, 0), 0)

    return pl.pallas_call(
        k,
        grid=(2, G),
        in_specs=[
            pl.BlockSpec((NC, BLK, N_FEAT), part_map),
            pl.BlockSpec((N_FEAT, N_FEAT), lambda t, i: (0, 0)),
            pl.BlockSpec((1, N_FEAT), lambda t, i: (0, 0)),
            pl.BlockSpec((1, N_FEAT), lambda t, i: (0, 0)),
            pl.BlockSpec((1, N_FEAT), lambda t, i: (0, 0)),
            pl.BlockSpec((N_FEAT, N_FEAT), lambda t, i: (0, 0)),
            pl.BlockSpec((1, N_FEAT), lambda t, i: (0, 0)),
            pl.BlockSpec((N_CLASS, N_FEAT), lambda t, i: (0, 0)),
            pl.BlockSpec((1, N_CLASS), lambda t, i: (0, 0)),
        ],
        out_specs=pl.BlockSpec((BLK, N_CLASS), lambda t, i: (i, 0)),
        out_shape=jax.ShapeDtypeStruct((N_NODES, N_CLASS), jnp.float32),
        scratch_shapes=[
            pltpu.VMEM((N_NODES, N_FEAT), jnp.float32),
            pltpu.VMEM((2, N_FEAT), jnp.float32),
        ],
    )(parts, w1t, b1, gamma, beta, w2t, b2, wfct, bfc)


def kernel(x, edge_index, W1, b1, gamma, beta, W2, b2, Wfc, bfc):
    idx6 = edge_index.astype(jnp.int32).reshape(2, NC, NS, NSEG, SEGCHUNK, CHUNK)
    parts = _sc_segment_sum(x, idx6)
    out = _tc_mlp(parts, W1, b1.reshape(1, N_FEAT),
                  gamma.reshape(1, N_FEAT), beta.reshape(1, N_FEAT),
                  W2, b2.reshape(1, N_FEAT), Wfc, bfc.reshape(1, N_CLASS))
    return out


# speedup vs baseline: 1.0206x; 1.0203x over previous
import functools

import jax
import jax.numpy as jnp
from jax import lax
from jax.experimental import pallas as pl
from jax.experimental.pallas import tpu as pltpu
from jax.experimental.pallas import tpu_sc as plsc

N_NODES = 10000
N_FEAT = 128
N_EDGES = 320000
N_CLASS = 40

NC = 2
NS = 16
EDGES_PER_TILE = N_EDGES // (NC * NS)
CHUNK = 80
NCHUNK = EDGES_PER_TILE // CHUNK
NSEG = 5
SEGCHUNK = NCHUNK // NSEG
ROWS_PER_TILE = 624
TAIL_ROWS = N_NODES - NS * ROWS_PER_TILE
ZROWS = 24


def _sc_segment_sum(x, idx6):
    mesh = plsc.VectorSubcoreMesh(core_axis_name="c", subcore_axis_name="s")

    @functools.partial(
        pl.kernel,
        mesh=mesh,
        compiler_params=pltpu.CompilerParams(use_tc_tiling_on_sc=False),
        out_type=jax.ShapeDtypeStruct((NC, N_NODES, N_FEAT), jnp.float32),
        scratch_types=[
            pltpu.VMEM((2, SEGCHUNK, CHUNK), jnp.int32),
            pltpu.VMEM((2, SEGCHUNK, CHUNK), jnp.int32),
            pltpu.VMEM((CHUNK, N_FEAT), jnp.float32),
            pltpu.VMEM((CHUNK, N_FEAT), jnp.float32),
            pltpu.VMEM((CHUNK, N_FEAT), jnp.float32),
            pltpu.VMEM_SHARED((N_NODES, N_FEAT), jnp.float32),
            pltpu.SemaphoreType.DMA,
            pltpu.SemaphoreType.DMA,
            pltpu.SemaphoreType.DMA,
            pltpu.SemaphoreType.DMA,
            pltpu.SemaphoreType.DMA,
        ],
    )
    def seg_kernel(x_hbm, idx_hbm, out_hbm,
                   srcv, dstv, bufa, bufb, bufc, aggs,
                   sema, semb, semc, semi0, semi1):
        c = lax.axis_index("c")
        s = lax.axis_index("s")
        row0 = s * ROWS_PER_TILE

        @pl.when(c == 0)
        def _():
            pltpu.sync_copy(x_hbm.at[pl.ds(row0, ROWS_PER_TILE)],
                            aggs.at[pl.ds(row0, ROWS_PER_TILE)])

            @pl.when(s == 0)
            def _():
                pltpu.sync_copy(x_hbm.at[pl.ds(NS * ROWS_PER_TILE, TAIL_ROWS)],
                                aggs.at[pl.ds(NS * ROWS_PER_TILE, TAIL_ROWS)])

        @pl.when(c == 1)
        def _():
            def zstore(k, carry):
                r = k // (N_FEAT // 16)
                col = (k % (N_FEAT // 16)) * 16
                bufa[r, pl.ds(col, 16)] = jnp.zeros((16,), jnp.float32)
                return carry
            lax.fori_loop(0, ZROWS * (N_FEAT // 16), zstore, 0)

            def zcopy(i, carry):
                pltpu.sync_copy(bufa.at[pl.ds(0, ZROWS)],
                                aggs.at[pl.ds(row0 + i * ZROWS, ZROWS)])
                return carry
            lax.fori_loop(0, ROWS_PER_TILE // ZROWS, zcopy, 0)

            @pl.when(s == 0)
            def _():
                pltpu.sync_copy(bufa.at[pl.ds(0, TAIL_ROWS)],
                                aggs.at[pl.ds(NS * ROWS_PER_TILE, TAIL_ROWS)])

        pltpu.sync_copy(idx_hbm.at[0, c, s, 0], srcv.at[0])
        pltpu.sync_copy(idx_hbm.at[1, c, s, 0], dstv.at[0])
        pltpu.make_async_copy(idx_hbm.at[0, c, s, 1], srcv.at[1], semi1).start()
        pltpu.make_async_copy(idx_hbm.at[1, c, s, 1], dstv.at[1], semi1).start()
        plsc.subcore_barrier()

        bufs = (bufa, bufb, bufc)
        sems = (sema, semb, semc)

        def seg(g, carry):
            p = g % 2
            sv = srcv.at[p]
            dv = dstv.at[p]

            @pl.when((g > 0) & (p == 0))
            def _():
                pltpu.make_async_copy(idx_hbm.at[0, c, s, g], sv, semi0).wait()
                pltpu.make_async_copy(idx_hbm.at[1, c, s, g], dv, semi0).wait()

            @pl.when((g > 0) & (p == 1))
            def _():
                pltpu.make_async_copy(idx_hbm.at[0, c, s, g], sv, semi1).wait()
                pltpu.make_async_copy(idx_hbm.at[1, c, s, g], dv, semi1).wait()

            pltpu.make_async_copy(x_hbm.at[sv.at[0]], bufs[0], sems[0]).start()
            pltpu.make_async_copy(x_hbm.at[sv.at[1]], bufs[1], sems[1]).start()
            for j in range(SEGCHUNK):
                b = j % 3
                pltpu.make_async_copy(x_hbm.at[sv.at[j]], bufs[b], sems[b]).wait()
                pltpu.make_async_copy(
                    bufs[b], aggs.at[dv.at[j]], sems[b]).start(add=True)
                if j + 2 < SEGCHUNK:
                    b2 = (j + 2) % 3
                    if j >= 1:
                        pltpu.make_async_copy(
                            bufs[b2], aggs.at[dv.at[j - 1]], sems[b2]).wait()
                    pltpu.make_async_copy(
                        x_hbm.at[sv.at[j + 2]], bufs[b2], sems[b2]).start()
            for j in range(SEGCHUNK - 3, SEGCHUNK):
                b = j % 3
                pltpu.make_async_copy(
                    bufs[b], aggs.at[dv.at[j]], sems[b]).wait()

            @pl.when((g + 2 < NSEG) & (p == 0))
            def _():
                pltpu.make_async_copy(idx_hbm.at[0, c, s, g + 2], srcv.at[p], semi0).start()
                pltpu.make_async_copy(idx_hbm.at[1, c, s, g + 2], dstv.at[p], semi0).start()

            @pl.when((g + 2 < NSEG) & (p == 1))
            def _():
                pltpu.make_async_copy(idx_hbm.at[0, c, s, g + 2], srcv.at[p], semi1).start()
                pltpu.make_async_copy(idx_hbm.at[1, c, s, g + 2], dstv.at[p], semi1).start()
            return carry
        lax.fori_loop(0, NSEG, seg, 0)

        plsc.subcore_barrier()
        pltpu.sync_copy(aggs.at[pl.ds(row0, ROWS_PER_TILE)],
                        out_hbm.at[c, pl.ds(row0, ROWS_PER_TILE)])

        @pl.when(s == 0)
        def _():
            pltpu.sync_copy(aggs.at[pl.ds(NS * ROWS_PER_TILE, TAIL_ROWS)],
                            out_hbm.at[c, pl.ds(NS * ROWS_PER_TILE, TAIL_ROWS)])

    return seg_kernel(x, idx6)


def _tc_mlp(parts, w1t, b1, gamma, beta, w2t, b2, wfct, bfc):
    BLK = 5000
    G = N_NODES // BLK
    inv_n = 1.0 / N_NODES

    def k(p_r, w1_r, b1_r, g_r, be_r, w2_r, b2_r, wf_r, bf_r,
          out_r, h1_s, sums_s):
        t = pl.program_id(0)
        i = pl.program_id(1)

        @pl.when(t == 0)
        def _():
            h = p_r[0] + p_r[1]
            h1 = lax.dot_general(h, w1_r[...], (((1,), (1,)), ((), ())),
                                 preferred_element_type=jnp.float32) + b1_r[...]
            h1 = jnp.maximum(h1, 0.0)
            h1_s[pl.ds(i * BLK, BLK), :] = h1

            @pl.when(i == 0)
            def _():
                sums_s[...] = jnp.zeros_like(sums_s)
            sums_s[0:1, :] += jnp.sum(h1, axis=0, keepdims=True)
            sums_s[1:2, :] += jnp.sum(h1 * h1, axis=0, keepdims=True)

        @pl.when(t == 1)
        def _():
            mean = sums_s[0:1, :] * inv_n
            var = sums_s[1:2, :] * inv_n - mean * mean
            sc = g_r[...] * lax.rsqrt(var + 1e-5)
            sh = be_r[...] - mean * sc
            ws = lax.dot_general(w2_r[...], wf_r[...], (((0,), (1,)), ((), ())),
                                 preferred_element_type=jnp.float32)
            bs = lax.dot_general(b2_r[...], wf_r[...], (((1,), (1,)), ((), ())),
                                 preferred_element_type=jnp.float32) + bf_r[...]
            hn = h1_s[pl.ds(i * BLK, BLK), :] * sc + sh
            out_r[...] = jnp.dot(hn, ws, preferred_element_type=jnp.float32) + bs

    def part_map(t, i):
        return (0, jnp.where(t == 0, i, 0), 0)

    return pl.pallas_call(
        k,
        grid=(2, G),
        in_specs=[
            pl.BlockSpec((NC, BLK, N_FEAT), part_map),
            pl.BlockSpec((N_FEAT, N_FEAT), lambda t, i: (0, 0)),
            pl.BlockSpec((1, N_FEAT), lambda t, i: (0, 0)),
            pl.BlockSpec((1, N_FEAT), lambda t, i: (0, 0)),
            pl.BlockSpec((1, N_FEAT), lambda t, i: (0, 0)),
            pl.BlockSpec((N_FEAT, N_FEAT), lambda t, i: (0, 0)),
            pl.BlockSpec((1, N_FEAT), lambda t, i: (0, 0)),
            pl.BlockSpec((N_CLASS, N_FEAT), lambda t, i: (0, 0)),
            pl.BlockSpec((1, N_CLASS), lambda t, i: (0, 0)),
        ],
        out_specs=pl.BlockSpec((BLK, N_CLASS), lambda t, i: (i, 0)),
        out_shape=jax.ShapeDtypeStruct((N_NODES, N_CLASS), jnp.float32),
        scratch_shapes=[
            pltpu.VMEM((N_NODES, N_FEAT), jnp.float32),
            pltpu.VMEM((2, N_FEAT), jnp.float32),
        ],
    )(parts, w1t, b1, gamma, beta, w2t, b2, wfct, bfc)


def kernel(x, edge_index, W1, b1, gamma, beta, W2, b2, Wfc, bfc):
    idx6 = edge_index.astype(jnp.int32).reshape(2, NC, NS, NSEG, SEGCHUNK, CHUNK)
    parts = _sc_segment_sum(x, idx6)
    out = _tc_mlp(parts, W1, b1.reshape(1, N_FEAT),
                  gamma.reshape(1, N_FEAT), beta.reshape(1, N_FEAT),
                  W2, b2.reshape(1, N_FEAT), Wfc, bfc.reshape(1, N_CLASS))
    return out
